# baseline (speedup 1.0000x reference)
import jax
import jax.numpy as jnp
from jax import lax
from jax.experimental import pallas as pl
from jax.experimental.pallas import tpu as pltpu

_D = 64
_N = 8
_OUT = 10
_R = 128
_CHAINS = 32
_GROUPS = 1
_TOTAL = _R * _CHAINS * _GROUPS

_CONV_ROWS = 352
_WDEC_R = 192
_BQ_R, _BK_R, _BV_R, _BCOMB_R = 320, 328, 336, 344
_W1_R = 2 * _CONV_ROWS
_W2_R = _W1_R + 3 * _D
_B1_R = _W2_R + 128
_B2_R = _B1_R + _N

_W4A, _W4B = 0, 64
_WBA, _WBB = 128, 192
_P_W1 = 256
_P_W2 = 448
_P_BIAS = 576
_WROWS = 584


def _pack_weights(slab):
    def pad256(a):
        return jnp.pad(a, ((0, 0), (0, 256 - a.shape[1])))

    def conv_parts(base):
        wq = slab[base + 0:base + 64, 0:_D]
        wk = slab[base + 64:base + 128, 0:_D]
        wv = slab[base + 128:base + 192, 0:_D]
        wtop = slab[base + _WDEC_R:base + _WDEC_R + _D, 0:_D]
        wbot = slab[base + _WDEC_R + _D:base + _WDEC_R + 2 * _D, 0:_D]
        w4 = jnp.concatenate([wq, wk, wv, wtop], axis=1)
        bias4 = jnp.concatenate(
            [slab[base + r, 0:_D] for r in (_BQ_R, _BK_R, _BV_R, _BCOMB_R)])
        return w4, pad256(wbot), bias4[None, :]

    w4_1, wbot_1, b4_1 = conv_parts(0)
    w4_2, wbot_2, b4_2 = conv_parts(_CONV_ROWS)
    w1 = pad256(slab[_W1_R:_W1_R + 3 * _D, :])
    w2 = pad256(slab[_W2_R:_W2_R + 128, :])
    b1 = pad256(slab[_B1_R:_B1_R + 1, :])
    b2 = pad256(slab[_B2_R:_B2_R + 1, :])
    bias_rows = jnp.concatenate(
        [b4_1, b4_2, b1, b2, jnp.zeros((4, 256), jnp.float32)], axis=0)
    return jnp.concatenate(
        [w4_1, w4_2, wbot_1, wbot_2, w1, w2, bias_rows], axis=0)


def _body(x_ref, w_ref, out_ref):
    f32 = jnp.float32

    r = lax.broadcasted_iota(jnp.int32, (_R, _R), 0)
    c = lax.broadcasted_iota(jnp.int32, (_R, _R), 1)
    mask = (r // _N) == (c // _N)

    def graph_conv(xin, w4_row, bias_idx):
        wbot_row = _WBA if w4_row == _W4A else _WBB
        w4 = w_ref[w4_row:w4_row + _D, :]
        bias = w_ref[_P_BIAS + bias_idx:_P_BIAS + bias_idx + 1, :]
        wbot = w_ref[wbot_row:wbot_row + _D, 0:_D]

        qkvt = [jnp.dot(xc, w4, preferred_element_type=f32) + bias
                for xc in xin]
        s = [lax.dot_general(t[:, 0:_D], t[:, _D:2 * _D],
                             (((1,), (1,)), ((), ())),
                             preferred_element_type=f32) for t in qkvt]
        s = [jnp.where(mask, sc, f32(-1e30)) for sc in s]
        m = [jnp.max(sc, axis=-1, keepdims=True) for sc in s]
        e = [jnp.exp(sc - mc) for sc, mc in zip(s, m)]
        attn = [ec / jnp.sum(ec, axis=-1, keepdims=True) for ec in e]
        ctx = [jnp.dot(ac, t[:, 2 * _D:3 * _D], preferred_element_type=f32)
               for ac, t in zip(attn, qkvt)]
        pre = [t[:, 3 * _D:4 * _D]
               + jnp.dot(cc, wbot, preferred_element_type=f32)
               for cc, t in zip(ctx, qkvt)]
        return [jnp.maximum(p, 0.0) for p in pre]

    w1 = w_ref[_P_W1:_P_W1 + 3 * _D, 0:128]
    b1 = w_ref[_P_BIAS + 2:_P_BIAS + 3, 0:128]
    w2 = w_ref[_P_W2:_P_W2 + 128, 0:_OUT]
    b2 = w_ref[_P_BIAS + 3:_P_BIAS + 4, 0:_OUT]

    gpc = _R // _N
    for gi in range(_GROUPS):
        base = gi * _CHAINS * gpc
        xs = [x_ref[base + ci * gpc:base + (ci + 1) * gpc].reshape(_R, _D)
              for ci in range(_CHAINS)]
        h1 = graph_conv(xs, _W4A, 0)
        h2 = graph_conv(h1, _W4B, 1)

        cat = [jnp.concatenate([xc, ha, hb], axis=1)
               for xc, ha, hb in zip(xs, h1, h2)]
        acc = [jnp.dot(cc, w1, preferred_element_type=f32) for cc in cat]
        a = [jnp.maximum(ac + b1, 0.0) for ac in acc]

        logits = [jnp.dot(ac, w2, preferred_element_type=f32) + b2 for ac in a]
        m = [jnp.max(lg, axis=-1, keepdims=True) for lg in logits]
        e = [jnp.exp(lg - mc) for lg, mc in zip(logits, m)]
        thresh = [0.1 * jnp.sum(ec, axis=-1, keepdims=True) for ec in e]
        for ci in range(_CHAINS):
            adj = jnp.where(e[ci] >= thresh[ci], 1.0,
                            0.0).astype(out_ref.dtype)
            out_ref[base + ci * gpc:base + (ci + 1) * gpc] = adj.reshape(
                gpc, _N, _OUT)


@jax.jit
def kernel(z_batch, slab):
    b = z_batch.shape[0]
    graphs_per_block = _TOTAL // _N
    b_pad = ((b + graphs_per_block - 1) // graphs_per_block) * graphs_per_block
    z = z_batch
    if b_pad != b:
        z = jnp.pad(z, ((0, b_pad - b), (0, 0), (0, 0)))
    rows = b_pad * _N
    wpack = _pack_weights(slab)

    flops_per_row = 2 * (64 * 256 + 64 * _R + _R * 64 + 64 * 64) * 2 \
        + 2 * (3 * 64 * 128 + 128 * _OUT)
    out = pl.pallas_call(
        _body,
        grid=(rows // _TOTAL,),
        in_specs=[
            pl.BlockSpec((graphs_per_block, _N, _D), lambda i: (i, 0, 0)),
            pl.BlockSpec((_WROWS, 256), lambda i: (0, 0)),
        ],
        out_specs=pl.BlockSpec((graphs_per_block, _N, _OUT),
                               lambda i: (i, 0, 0)),
        out_shape=jax.ShapeDtypeStruct((b_pad, _N, _OUT), jnp.float32),
        compiler_params=pltpu.CompilerParams(
            dimension_semantics=("parallel",)),
        cost_estimate=pl.CostEstimate(
            flops=rows * flops_per_row,
            transcendentals=rows * (_R + _OUT),
            bytes_accessed=_WROWS * 256 * 4 + rows * (_D + _OUT) * 4),
    )(z, wpack)
    return out[:b]


# speedup vs baseline: 1.0393x; 1.0393x over previous
import jax
import jax.numpy as jnp
from jax import lax
from jax.experimental import pallas as pl
from jax.experimental.pallas import tpu as pltpu

_D = 64
_N = 8
_OUT = 10
_R = 128
_CHAINS = 32
_GROUPS = 1
_TOTAL = _R * _CHAINS * _GROUPS

_CONV_ROWS = 352
_WDEC_R = 192
_BQ_R, _BK_R, _BV_R, _BCOMB_R = 320, 328, 336, 344
_W1_R = 2 * _CONV_ROWS
_W2_R = _W1_R + 3 * _D
_B1_R = _W2_R + 128
_B2_R = _B1_R + _N

_W4A, _W4B = 0, 64
_WBA, _WBB = 128, 192
_P_W1 = 256
_P_W2 = 448
_P_BIAS = 576
_WROWS = 584


def _pack_weights(slab):
    def pad256(a):
        return jnp.pad(a, ((0, 0), (0, 256 - a.shape[1])))

    def conv_parts(base):
        wq = slab[base + 0:base + 64, 0:_D]
        wk = slab[base + 64:base + 128, 0:_D]
        wv = slab[base + 128:base + 192, 0:_D]
        wtop = slab[base + _WDEC_R:base + _WDEC_R + _D, 0:_D]
        wbot = slab[base + _WDEC_R + _D:base + _WDEC_R + 2 * _D, 0:_D]
        w4 = jnp.concatenate([wq, wk, wv, wtop], axis=1)
        bias4 = jnp.concatenate(
            [slab[base + r, 0:_D] for r in (_BQ_R, _BK_R, _BV_R, _BCOMB_R)])
        return w4, pad256(wbot), bias4[None, :]

    w4_1, wbot_1, b4_1 = conv_parts(0)
    w4_2, wbot_2, b4_2 = conv_parts(_CONV_ROWS)
    w1 = pad256(slab[_W1_R:_W1_R + 3 * _D, :])
    w2 = pad256(slab[_W2_R:_W2_R + 128, :])
    b1 = pad256(slab[_B1_R:_B1_R + 1, :])
    b2 = pad256(slab[_B2_R:_B2_R + 1, :])
    bias_rows = jnp.concatenate(
        [b4_1, b4_2, b1, b2, jnp.zeros((4, 256), jnp.float32)], axis=0)
    return jnp.concatenate(
        [w4_1, w4_2, wbot_1, wbot_2, w1, w2, bias_rows], axis=0)


def _body(x_ref, w_ref, out_ref):
    f32 = jnp.float32

    r = lax.broadcasted_iota(jnp.int32, (_R, _R), 0)
    c = lax.broadcasted_iota(jnp.int32, (_R, _R), 1)
    mask = (r // _N) == (c // _N)

    def graph_conv(xin, w4_row, bias_idx):
        wbot_row = _WBA if w4_row == _W4A else _WBB
        w4 = w_ref[w4_row:w4_row + _D, :]
        bias = w_ref[_P_BIAS + bias_idx:_P_BIAS + bias_idx + 1, :]
        wbot = w_ref[wbot_row:wbot_row + _D, 0:_D]

        qkvt = [jnp.dot(xc, w4, preferred_element_type=f32) + bias
                for xc in xin]
        s = [lax.dot_general(t[:, 0:_D], t[:, _D:2 * _D],
                             (((1,), (1,)), ((), ())),
                             preferred_element_type=f32) for t in qkvt]
        s = [jnp.where(mask, sc, f32(-1e30)) for sc in s]
        m = [jnp.max(sc, axis=-1, keepdims=True) for sc in s]
        e = [jnp.exp(sc - mc) for sc, mc in zip(s, m)]
        attn = [ec / jnp.sum(ec, axis=-1, keepdims=True) for ec in e]
        ctx = [jnp.dot(ac, t[:, 2 * _D:3 * _D], preferred_element_type=f32)
               for ac, t in zip(attn, qkvt)]
        pre = [t[:, 3 * _D:4 * _D]
               + jnp.dot(cc, wbot, preferred_element_type=f32)
               for cc, t in zip(ctx, qkvt)]
        return [jnp.maximum(p, 0.0) for p in pre]

    w1 = w_ref[_P_W1:_P_W1 + 3 * _D, 0:128]
    b1 = w_ref[_P_BIAS + 2:_P_BIAS + 3, 0:128]
    w2 = w_ref[_P_W2:_P_W2 + 128, 0:_OUT]
    b2 = w_ref[_P_BIAS + 3:_P_BIAS + 4, 0:_OUT]

    gpc = _R // _N
    for gi in range(_GROUPS):
        base = gi * _CHAINS * gpc
        xs = [x_ref[gi * _CHAINS * _R + ci * _R:
                    gi * _CHAINS * _R + (ci + 1) * _R, :]
              for ci in range(_CHAINS)]
        h1 = graph_conv(xs, _W4A, 0)
        h2 = graph_conv(h1, _W4B, 1)

        cat = [jnp.concatenate([xc, ha, hb], axis=1)
               for xc, ha, hb in zip(xs, h1, h2)]
        acc = [jnp.dot(cc, w1, preferred_element_type=f32) for cc in cat]
        a = [jnp.maximum(ac + b1, 0.0) for ac in acc]

        logits = [jnp.dot(ac, w2, preferred_element_type=f32) + b2 for ac in a]
        m = [jnp.max(lg, axis=-1, keepdims=True) for lg in logits]
        e = [jnp.exp(lg - mc) for lg, mc in zip(logits, m)]
        thresh = [0.1 * jnp.sum(ec, axis=-1, keepdims=True) for ec in e]
        for ci in range(_CHAINS):
            adj = jnp.where(e[ci] >= thresh[ci], 1.0,
                            0.0).astype(out_ref.dtype)
            out_ref[base + ci * gpc:base + (ci + 1) * gpc] = adj.reshape(
                gpc, _N, _OUT)


@jax.jit
def kernel(z_batch, slab):
    b = z_batch.shape[0]
    graphs_per_block = _TOTAL // _N
    b_pad = ((b + graphs_per_block - 1) // graphs_per_block) * graphs_per_block
    z = z_batch
    if b_pad != b:
        z = jnp.pad(z, ((0, b_pad - b), (0, 0), (0, 0)))
    rows = b_pad * _N
    x = z.reshape(rows, _D)
    wpack = _pack_weights(slab)

    flops_per_row = 2 * (64 * 256 + 64 * _R + _R * 64 + 64 * 64) * 2 \
        + 2 * (3 * 64 * 128 + 128 * _OUT)
    out = pl.pallas_call(
        _body,
        grid=(rows // _TOTAL,),
        in_specs=[
            pl.BlockSpec((_TOTAL, _D), lambda i: (i, 0)),
            pl.BlockSpec((_WROWS, 256), lambda i: (0, 0)),
        ],
        out_specs=pl.BlockSpec((graphs_per_block, _N, _OUT),
                               lambda i: (i, 0, 0)),
        out_shape=jax.ShapeDtypeStruct((b_pad, _N, _OUT), jnp.float32),
        compiler_params=pltpu.CompilerParams(
            dimension_semantics=("parallel",)),
        cost_estimate=pl.CostEstimate(
            flops=rows * flops_per_row,
            transcendentals=rows * (_R + _OUT),
            bytes_accessed=_WROWS * 256 * 4 + rows * (_D + _OUT) * 4),
    )(x, wpack)
    return out[:b]


# speedup vs baseline: 1.1388x; 1.0957x over previous
import jax
import jax.numpy as jnp
from jax import lax
from jax.experimental import pallas as pl
from jax.experimental.pallas import tpu as pltpu

_D = 64
_N = 8
_OUT = 10
_R = 128
_CHAINS = 32
_GROUPS = 1
_TOTAL = _R * _CHAINS * _GROUPS

_CONV_ROWS = 352
_WDEC_R = 192
_BQ_R, _BK_R, _BV_R, _BCOMB_R = 320, 328, 336, 344
_W1_R = 2 * _CONV_ROWS
_W2_R = _W1_R + 3 * _D
_B1_R = _W2_R + 128
_B2_R = _B1_R + _N

_W4A, _W4B = 0, 64
_WBA, _WBB = 128, 192
_P_W1 = 256
_P_W2 = 448
_P_BIAS = 576
_WROWS = 584


def _pack_weights(slab):
    def pad256(a):
        return jnp.pad(a, ((0, 0), (0, 256 - a.shape[1])))

    def conv_parts(base):
        wq = slab[base + 0:base + 64, 0:_D]
        wk = slab[base + 64:base + 128, 0:_D]
        wv = slab[base + 128:base + 192, 0:_D]
        wtop = slab[base + _WDEC_R:base + _WDEC_R + _D, 0:_D]
        wbot = slab[base + _WDEC_R + _D:base + _WDEC_R + 2 * _D, 0:_D]
        w4 = jnp.concatenate([wq, wk, wv, wtop], axis=1)
        bias4 = jnp.concatenate(
            [slab[base + r, 0:_D] for r in (_BQ_R, _BK_R, _BV_R, _BCOMB_R)])
        return w4, pad256(wbot), bias4[None, :]

    w4_1, wbot_1, b4_1 = conv_parts(0)
    w4_2, wbot_2, b4_2 = conv_parts(_CONV_ROWS)
    w1 = pad256(slab[_W1_R:_W1_R + 3 * _D, :])
    w2 = pad256(slab[_W2_R:_W2_R + 128, :])
    b1 = pad256(slab[_B1_R:_B1_R + 1, :])
    b2 = pad256(slab[_B2_R:_B2_R + 1, :])
    bias_rows = jnp.concatenate(
        [b4_1, b4_2, b1, b2, jnp.zeros((4, 256), jnp.float32)], axis=0)
    return jnp.concatenate(
        [w4_1, w4_2, wbot_1, wbot_2, w1, w2, bias_rows], axis=0)


def _body(x_ref, w_ref, out_ref):
    f32 = jnp.float32

    r = lax.broadcasted_iota(jnp.int32, (_R, _R), 0)
    c = lax.broadcasted_iota(jnp.int32, (_R, _R), 1)
    mask = (r // _N) == (c // _N)

    def graph_conv(xin, w4_row, bias_idx):
        wbot_row = _WBA if w4_row == _W4A else _WBB
        w4 = w_ref[w4_row:w4_row + _D, :]
        bias = w_ref[_P_BIAS + bias_idx:_P_BIAS + bias_idx + 1, :]
        wbot = w_ref[wbot_row:wbot_row + _D, 0:_D]

        qkvt = [jnp.dot(xc, w4, preferred_element_type=f32) + bias
                for xc in xin]
        s = [lax.dot_general(t[:, 0:_D], t[:, _D:2 * _D],
                             (((1,), (1,)), ((), ())),
                             preferred_element_type=f32) for t in qkvt]
        s = [jnp.where(mask, sc, f32(-1e30)) for sc in s]
        e = [jnp.exp(sc) for sc in s]
        denom = [jnp.sum(ec, axis=-1, keepdims=True) for ec in e]
        ctx = [jnp.dot(ec, t[:, 2 * _D:3 * _D], preferred_element_type=f32)
               / dc for ec, t, dc in zip(e, qkvt, denom)]
        pre = [t[:, 3 * _D:4 * _D]
               + jnp.dot(cc, wbot, preferred_element_type=f32)
               for cc, t in zip(ctx, qkvt)]
        return [jnp.maximum(p, 0.0) for p in pre]

    w1 = w_ref[_P_W1:_P_W1 + 3 * _D, 0:128]
    b1 = w_ref[_P_BIAS + 2:_P_BIAS + 3, 0:128]
    w2 = w_ref[_P_W2:_P_W2 + 128, 0:_OUT]
    b2 = w_ref[_P_BIAS + 3:_P_BIAS + 4, 0:_OUT]

    gpc = _R // _N
    for gi in range(_GROUPS):
        base = gi * _CHAINS * gpc
        xs = [x_ref[gi * _CHAINS * _R + ci * _R:
                    gi * _CHAINS * _R + (ci + 1) * _R, :]
              for ci in range(_CHAINS)]
        h1 = graph_conv(xs, _W4A, 0)
        h2 = graph_conv(h1, _W4B, 1)

        cat = [jnp.concatenate([xc, ha, hb], axis=1)
               for xc, ha, hb in zip(xs, h1, h2)]
        acc = [jnp.dot(cc, w1, preferred_element_type=f32) for cc in cat]
        a = [jnp.maximum(ac + b1, 0.0) for ac in acc]

        logits = [jnp.dot(ac, w2, preferred_element_type=f32) + b2 for ac in a]
        m = [jnp.max(lg, axis=-1, keepdims=True) for lg in logits]
        e = [jnp.exp(lg - mc) for lg, mc in zip(logits, m)]
        thresh = [0.1 * jnp.sum(ec, axis=-1, keepdims=True) for ec in e]
        for ci in range(_CHAINS):
            adj = jnp.where(e[ci] >= thresh[ci], 1.0,
                            0.0).astype(out_ref.dtype)
            out_ref[base + ci * gpc:base + (ci + 1) * gpc] = adj.reshape(
                gpc, _N, _OUT)


@jax.jit
def kernel(z_batch, slab):
    b = z_batch.shape[0]
    graphs_per_block = _TOTAL // _N
    b_pad = ((b + graphs_per_block - 1) // graphs_per_block) * graphs_per_block
    z = z_batch
    if b_pad != b:
        z = jnp.pad(z, ((0, b_pad - b), (0, 0), (0, 0)))
    rows = b_pad * _N
    x = z.reshape(rows, _D)
    wpack = _pack_weights(slab)

    flops_per_row = 2 * (64 * 256 + 64 * _R + _R * 64 + 64 * 64) * 2 \
        + 2 * (3 * 64 * 128 + 128 * _OUT)
    out = pl.pallas_call(
        _body,
        grid=(rows // _TOTAL,),
        in_specs=[
            pl.BlockSpec((_TOTAL, _D), lambda i: (i, 0)),
            pl.BlockSpec((_WROWS, 256), lambda i: (0, 0)),
        ],
        out_specs=pl.BlockSpec((graphs_per_block, _N, _OUT),
                               lambda i: (i, 0, 0)),
        out_shape=jax.ShapeDtypeStruct((b_pad, _N, _OUT), jnp.float32),
        compiler_params=pltpu.CompilerParams(
            dimension_semantics=("parallel",)),
        cost_estimate=pl.CostEstimate(
            flops=rows * flops_per_row,
            transcendentals=rows * (_R + _OUT),
            bytes_accessed=_WROWS * 256 * 4 + rows * (_D + _OUT) * 4),
    )(x, wpack)
    return out[:b]
